# baseline (speedup 1.0000x reference)
import jax
import jax.numpy as jnp
from jax.experimental import pallas as pl
from jax.experimental.pallas import tpu as pltpu

_TM = 1024


def _fused_t_body(x_ref, labt_ref, wt_ref, b_ref, ot_ref, labt_o_ref):
    ot_ref[...] = (
        jax.lax.dot_general(
            wt_ref[...], x_ref[...],
            dimension_numbers=(((1,), (1,)), ((), ())),
            preferred_element_type=jnp.float32,
        )
        + b_ref[...]
    )

    @pl.when(pl.program_id(0) == 0)
    def _():
        labt_o_ref[...] = labt_ref[...].astype(jnp.float32)


def _mlp_t_body(x_ref, w_ref, b_ref, ot_ref):
    ot_ref[...] = (
        jax.lax.dot_general(
            w_ref[...], x_ref[...],
            dimension_numbers=(((0,), (1,)), ((), ())),
            preferred_element_type=jnp.float32,
        )
        + b_ref[...]
    )


def kernel(pooled_features, labels, weight, bias):
    b, g, d = pooled_features.shape
    out = weight.shape[1]
    n = b * g

    flat = pooled_features.reshape(n, d).astype(jnp.float32)
    w = weight.astype(jnp.float32)
    bias_col = bias.astype(jnp.float32).reshape(out, 1)

    tm = min(_TM, n)
    grid = (pl.cdiv(n, tm),)

    if n % tm == 0:
        feats_t, labt = pl.pallas_call(
            _fused_t_body,
            out_shape=(
                jax.ShapeDtypeStruct((out, n), jnp.float32),
                jax.ShapeDtypeStruct((g, b), jnp.float32),
            ),
            grid=grid,
            in_specs=[
                pl.BlockSpec((tm, d), lambda i: (i, 0)),
                pl.BlockSpec((g, b), lambda i: (0, 0)),
                pl.BlockSpec((out, d), lambda i: (0, 0)),
                pl.BlockSpec((out, 1), lambda i: (0, 0)),
            ],
            out_specs=(
                pl.BlockSpec((out, tm), lambda i: (0, i)),
                pl.BlockSpec((g, b), lambda i: (0, 0)),
            ),
            compiler_params=pltpu.CompilerParams(
                dimension_semantics=("arbitrary",),
            ),
        )(flat, labels.T, w.T, bias_col)
        return feats_t.T, labt.T

    feats_t = pl.pallas_call(
        _mlp_t_body,
        out_shape=jax.ShapeDtypeStruct((out, n), jnp.float32),
        grid=grid,
        in_specs=[
            pl.BlockSpec((tm, d), lambda i: (i, 0)),
            pl.BlockSpec((d, out), lambda i: (0, 0)),
            pl.BlockSpec((out, 1), lambda i: (0, 0)),
        ],
        out_specs=pl.BlockSpec((out, tm), lambda i: (0, i)),
        compiler_params=pltpu.CompilerParams(
            dimension_semantics=("parallel",),
        ),
    )(flat, w, bias_col)
    return feats_t.T, labels.astype(jnp.float32)


# speedup vs baseline: 1.1967x; 1.1967x over previous
import jax
import jax.numpy as jnp
from jax.experimental import pallas as pl
from jax.experimental.pallas import tpu as pltpu

_TM = 2048


def _fused_t_body(x_ref, labt_ref, wt_ref, b_ref, ot_ref, labt_o_ref):
    ot_ref[...] = (
        jax.lax.dot_general(
            wt_ref[...], x_ref[...],
            dimension_numbers=(((1,), (1,)), ((), ())),
            preferred_element_type=jnp.float32,
        )
        + b_ref[...]
    )

    @pl.when(pl.program_id(0) == 0)
    def _():
        labt_o_ref[...] = labt_ref[...].astype(jnp.float32)


def _mlp_t_body(x_ref, w_ref, b_ref, ot_ref):
    ot_ref[...] = (
        jax.lax.dot_general(
            w_ref[...], x_ref[...],
            dimension_numbers=(((0,), (1,)), ((), ())),
            preferred_element_type=jnp.float32,
        )
        + b_ref[...]
    )


def kernel(pooled_features, labels, weight, bias):
    b, g, d = pooled_features.shape
    out = weight.shape[1]
    n = b * g

    flat = pooled_features.reshape(n, d).astype(jnp.float32)
    w = weight.astype(jnp.float32)
    bias_col = bias.astype(jnp.float32).reshape(out, 1)

    tm = min(_TM, n)
    grid = (pl.cdiv(n, tm),)

    if n % tm == 0:
        feats_t, labt = pl.pallas_call(
            _fused_t_body,
            out_shape=(
                jax.ShapeDtypeStruct((out, n), jnp.float32),
                jax.ShapeDtypeStruct((g, b), jnp.float32),
            ),
            grid=grid,
            in_specs=[
                pl.BlockSpec((tm, d), lambda i: (i, 0)),
                pl.BlockSpec((g, b), lambda i: (0, 0)),
                pl.BlockSpec((out, d), lambda i: (0, 0)),
                pl.BlockSpec((out, 1), lambda i: (0, 0)),
            ],
            out_specs=(
                pl.BlockSpec((out, tm), lambda i: (0, i)),
                pl.BlockSpec((g, b), lambda i: (0, 0)),
            ),
            compiler_params=pltpu.CompilerParams(
                dimension_semantics=("arbitrary",),
            ),
        )(flat, labels.T, w.T, bias_col)
        return feats_t.T, labt.T

    feats_t = pl.pallas_call(
        _mlp_t_body,
        out_shape=jax.ShapeDtypeStruct((out, n), jnp.float32),
        grid=grid,
        in_specs=[
            pl.BlockSpec((tm, d), lambda i: (i, 0)),
            pl.BlockSpec((d, out), lambda i: (0, 0)),
            pl.BlockSpec((out, 1), lambda i: (0, 0)),
        ],
        out_specs=pl.BlockSpec((out, tm), lambda i: (0, i)),
        compiler_params=pltpu.CompilerParams(
            dimension_semantics=("parallel",),
        ),
    )(flat, w, bias_col)
    return feats_t.T, labels.astype(jnp.float32)
